# baseline (speedup 1.0000x reference)
import functools

import jax
import jax.numpy as jnp
from jax.experimental import pallas as pl
from jax.experimental.pallas import tpu as pltpu
from jax.experimental.pallas import tpu_sc as plsc

_CONF_THRES = 0.4
_IOU_THRES = 0.45
_MAX_DET = 1000
_MAX_WH = 4096.0
_N = 20000
_NPAD = 20480
_R = _NPAD // 128
_NC = 80

_NW = 16
_CHUNK = _NPAD // _NW
_C = 256
_BUCK = _C * 128
_BUCKPAD = _BUCK + 16
_DUMPB = _BUCK
_BSL = _BUCK // _NW
_CM = 1024
_CMPAD = _CM + 8
_DUMPC = _CM
_CSH = _CM // _NW
_DETROWS = 1008
_DETFLAT = _DETROWS * 128
_DETSL = _DETFLAT // _NW
_DUMPD = _MAX_DET * 128


def _score_boxes(pt_ref):
    cx = pt_ref[0]
    cy = pt_ref[1]
    w = pt_ref[2]
    h = pt_ref[3]
    obj = pt_ref[4]
    best = pt_ref[5] * obj
    bidx = jnp.zeros((_R, 128), jnp.float32)
    for c in range(1, _NC):
        v = pt_ref[5 + c] * obj
        upd = v > best
        best = jnp.where(upd, v, best)
        bidx = jnp.where(upd, jnp.float32(c), bidx)
    valid = best > _CONF_THRES
    s0 = jnp.where(valid, best, -1e9)
    bx1 = cx - w / 2.0
    by1 = cy - h / 2.0
    bx2 = cx + w / 2.0
    by2 = cy + h / 2.0
    off = bidx * _MAX_WH
    return s0, valid, bidx, bx1, by1, bx2, by2, off


_CH_S = 0
_CH_NX1 = 1
_CH_NY1 = 2
_CH_NX2 = 3
_CH_NY2 = 4
_CH_AREA = 5
_CH_CLS = 6
_CH_BX1 = 7
_CH_BY1 = 8
_CH_BX2 = 9
_CH_BY2 = 10
_CH_LIN = 11
_NCH = 12


def _nms_full_kernel(pt_ref, out_ref, ch_ref):
    s0, valid, bidx, bx1, by1, bx2, by2, off = _score_boxes(pt_ref)
    nx1 = bx1 + off
    ny1 = by1 + off
    nx2 = bx2 + off
    ny2 = by2 + off
    areas = (nx2 - nx1) * (ny2 - ny1)
    lin = (jax.lax.broadcasted_iota(jnp.int32, (_R, 128), 0) * 128
           + jax.lax.broadcasted_iota(jnp.int32, (_R, 128), 1)
           ).astype(jnp.float32)
    ch_ref[_CH_S] = s0
    ch_ref[_CH_NX1] = nx1
    ch_ref[_CH_NY1] = ny1
    ch_ref[_CH_NX2] = nx2
    ch_ref[_CH_NY2] = ny2
    ch_ref[_CH_AREA] = areas
    ch_ref[_CH_CLS] = bidx
    ch_ref[_CH_BX1] = bx1
    ch_ref[_CH_BY1] = by1
    ch_ref[_CH_BX2] = bx2
    ch_ref[_CH_BY2] = by2
    ch_ref[_CH_LIN] = lin
    lane = jax.lax.broadcasted_iota(jnp.int32, (1, 128), 1).astype(jnp.float32)

    def body(i, _):
        s = ch_ref[_CH_S]
        linv = ch_ref[_CH_LIN]
        m = jnp.max(s)
        ok = m > -1e8
        r = jnp.where(s >= m, linv, 3.0e7)
        idxf = jnp.min(r)
        ii = idxf.astype(jnp.int32)
        ri = jax.lax.shift_right_logical(ii, 7)
        ci = (ii & 127).astype(jnp.float32)

        def gat(chan):
            row = ch_ref[chan, pl.ds(ri, 1), :]
            return jnp.sum(jnp.where(lane == ci, row, 0.0))

        gx1 = gat(_CH_BX1)
        gy1 = gat(_CH_BY1)
        gx2 = gat(_CH_BX2)
        gy2 = gat(_CH_BY2)
        gcls = gat(_CH_CLS)
        goff = gcls * _MAX_WH
        bnx1 = gx1 + goff
        bny1 = gy1 + goff
        bnx2 = gx2 + goff
        bny2 = gy2 + goff
        barea = (bnx2 - bnx1) * (bny2 - bny1)
        x1 = jnp.maximum(ch_ref[_CH_NX1], bnx1)
        y1 = jnp.maximum(ch_ref[_CH_NY1], bny1)
        x2 = jnp.minimum(ch_ref[_CH_NX2], bnx2)
        y2 = jnp.minimum(ch_ref[_CH_NY2], bny2)
        inter = jnp.maximum(x2 - x1, 0.0) * jnp.maximum(y2 - y1, 0.0)
        iou = inter / (ch_ref[_CH_AREA] + barea - inter + 1e-9)
        sup = (iou > _IOU_THRES) & ok
        hit = r == idxf
        ch_ref[_CH_S] = jnp.where(sup | hit, -1e9, s)
        okf = jnp.where(ok, 1.0, 0.0)
        row = jnp.where(lane == 0.0, gx1,
              jnp.where(lane == 1.0, gy1,
              jnp.where(lane == 2.0, gx2,
              jnp.where(lane == 3.0, gy2,
              jnp.where(lane == 4.0, m,
              jnp.where(lane == 5.0, gcls, 0.0))))))
        out_ref[pl.ds(i, 1), :] = row * okf
        return 0

    jax.lax.fori_loop(0, _MAX_DET, body, 0)


def _run_full(pt):
    out = pl.pallas_call(
        _nms_full_kernel,
        out_shape=jax.ShapeDtypeStruct((_MAX_DET, 128), jnp.float32),
        scratch_shapes=[pltpu.VMEM((_NCH, _R, 128), jnp.float32)],
    )(pt)
    return out[:, :6]


def _prep_kernel(pt_ref, s_ref, cls_ref, x1_ref, y1_ref, x2_ref, y2_ref):
    s0, valid, bidx, bx1, by1, bx2, by2, off = _score_boxes(pt_ref)
    s_ref[...] = s0
    cls_ref[...] = jnp.where(valid, bidx.astype(jnp.int32), 127)
    x1_ref[...] = bx1 + off
    y1_ref[...] = by1 + off
    x2_ref[...] = bx2 + off
    y2_ref[...] = by2 + off


def _run_prep(pt):
    sh_f = jax.ShapeDtypeStruct((_R, 128), jnp.float32)
    sh_i = jax.ShapeDtypeStruct((_R, 128), jnp.int32)
    return pl.pallas_call(
        _prep_kernel,
        out_shape=(sh_f, sh_i, sh_f, sh_f, sh_f, sh_f),
    )(pt)


_IOTA16 = None


def _sc_ext(ref, i, zero):
    g = (i // 16) * 16
    v = ref[pl.ds(g, 16)]
    io = jax.lax.broadcasted_iota(jnp.int32, (16,), 0)
    return jnp.sum(jnp.where(io == i - g, v, zero))


def _sc_put(ref, i, val, dtype):
    io = jax.lax.broadcasted_iota(jnp.int32, (16,), 0)
    idxv = jnp.zeros((16,), jnp.int32) + i
    xv = jnp.zeros((16,), dtype) + val
    plsc.store_scatter(ref, [idxv], xv, mask=io == 0)


def _sc_bucket_kernel(s_hbm, cls_hbm, x1_hbm, y1_hbm, x2_hbm, y2_hbm,
                      bs_hbm, bx1_hbm, by1_hbm, bx2_hbm, by2_hbm, blin_hbm,
                      whist_hbm,
                      s_v, cls_v, x1_v, y1_v, x2_v, y2_v, lin_v, slots_v,
                      hist_v, base_v, allhist_v, fill_f, fill_i,
                      shared_hist, sem):
    wid = jax.lax.axis_index("s")
    cbase = wid * _CHUNK
    iota16 = jax.lax.broadcasted_iota(jnp.int32, (16,), 0)

    pltpu.sync_copy(s_hbm.at[pl.ds(cbase, _CHUNK)], s_v)
    pltpu.sync_copy(cls_hbm.at[pl.ds(cbase, _CHUNK)], cls_v)
    pltpu.sync_copy(x1_hbm.at[pl.ds(cbase, _CHUNK)], x1_v)
    pltpu.sync_copy(y1_hbm.at[pl.ds(cbase, _CHUNK)], y1_v)
    pltpu.sync_copy(x2_hbm.at[pl.ds(cbase, _CHUNK)], x2_v)
    pltpu.sync_copy(y2_hbm.at[pl.ds(cbase, _CHUNK)], y2_v)

    ob = wid * _BSL
    for j in range(_BSL // 16):
        fill_f[pl.ds(j * 16, 16)] = jnp.full((16,), -1e9, jnp.float32)
        fill_i[pl.ds(j * 16, 16)] = jnp.zeros((16,), jnp.int32)
    pltpu.sync_copy(fill_f, bs_hbm.at[pl.ds(ob, _BSL)])
    pltpu.sync_copy(fill_i, blin_hbm.at[pl.ds(ob, _BSL)])
    for j in range(_BSL // 16):
        fill_f[pl.ds(j * 16, 16)] = jnp.zeros((16,), jnp.float32)
    pltpu.sync_copy(fill_f, bx1_hbm.at[pl.ds(ob, _BSL)])
    pltpu.sync_copy(fill_f, by1_hbm.at[pl.ds(ob, _BSL)])
    pltpu.sync_copy(fill_f, bx2_hbm.at[pl.ds(ob, _BSL)])
    pltpu.sync_copy(fill_f, by2_hbm.at[pl.ds(ob, _BSL)])

    for j in range(8):
        hist_v[pl.ds(j * 16, 16)] = jnp.zeros((16,), jnp.int32)

    def hbody(i, c0):
        c = _sc_ext(cls_v, i, 0)
        h = _sc_ext(hist_v, c, 0)
        _sc_put(hist_v, c, h + 1, jnp.int32)
        return c0

    jax.lax.fori_loop(0, _CHUNK, hbody, jnp.int32(0))

    pltpu.sync_copy(hist_v, whist_hbm.at[pl.ds(wid * 128, 128)])
    pltpu.sync_copy(hist_v, shared_hist.at[pl.ds(wid * 128, 128)])
    plsc.subcore_barrier()
    pltpu.sync_copy(shared_hist, allhist_v)

    for j in range(8):
        base_v[pl.ds(j * 16, 16)] = jnp.zeros((16,), jnp.int32)
    for w in range(_NW):
        gate = (wid > w).astype(jnp.int32)
        for j in range(8):
            row = allhist_v[pl.ds(w * 128 + j * 16, 16)]
            base_v[pl.ds(j * 16, 16)] = base_v[pl.ds(j * 16, 16)] + row * gate

    def abody(i, c0):
        c = _sc_ext(cls_v, i, 0)
        r = _sc_ext(base_v, c, 0)
        _sc_put(base_v, c, r + 1, jnp.int32)
        slot = jnp.where(c == 127, _DUMPB,
                         jnp.minimum(r, _C - 1) * 128 + c)
        io = jax.lax.broadcasted_iota(jnp.int32, (16,), 0)
        rowv = jnp.zeros((16,), jnp.int32) + i // 128
        colv = jnp.zeros((16,), jnp.int32) + i % 128
        xv = jnp.zeros((16,), jnp.int32) + slot
        plsc.store_scatter(slots_v, [rowv, colv], xv, mask=io == 0)
        return c0

    jax.lax.fori_loop(0, _CHUNK, abody, jnp.int32(0))

    for j in range(_CHUNK // 16):
        lin_v[pl.ds(j * 16, 16)] = cbase + j * 16 + iota16

    copies = []
    for k in range(_CHUNK // 128):
        idx = slots_v.at[k]
        sl = pl.ds(k * 128, 128)
        copies.append(pltpu.async_copy(s_v.at[sl], bs_hbm.at[idx], sem))
        copies.append(pltpu.async_copy(x1_v.at[sl], bx1_hbm.at[idx], sem))
        copies.append(pltpu.async_copy(y1_v.at[sl], by1_hbm.at[idx], sem))
        copies.append(pltpu.async_copy(x2_v.at[sl], bx2_hbm.at[idx], sem))
        copies.append(pltpu.async_copy(y2_v.at[sl], by2_hbm.at[idx], sem))
        copies.append(pltpu.async_copy(lin_v.at[sl], blin_hbm.at[idx], sem))
    for cp in copies:
        cp.wait()


def _run_bucket(s, cls, x1, y1, x2, y2):
    mesh = plsc.VectorSubcoreMesh(core_axis_name="c", subcore_axis_name="s",
                                  num_cores=1)
    f = jnp.float32
    i = jnp.int32
    kern = functools.partial(
        pl.kernel,
        mesh=mesh,
        compiler_params=pltpu.CompilerParams(needs_layout_passes=False),
        out_type=(
            jax.ShapeDtypeStruct((_BUCKPAD,), f),
            jax.ShapeDtypeStruct((_BUCKPAD,), f),
            jax.ShapeDtypeStruct((_BUCKPAD,), f),
            jax.ShapeDtypeStruct((_BUCKPAD,), f),
            jax.ShapeDtypeStruct((_BUCKPAD,), f),
            jax.ShapeDtypeStruct((_BUCKPAD,), i),
            jax.ShapeDtypeStruct((_NW * 128,), i),
        ),
        scratch_types=[
            pltpu.VMEM((_CHUNK,), f),
            pltpu.VMEM((_CHUNK,), i),
            pltpu.VMEM((_CHUNK,), f),
            pltpu.VMEM((_CHUNK,), f),
            pltpu.VMEM((_CHUNK,), f),
            pltpu.VMEM((_CHUNK,), f),
            pltpu.VMEM((_CHUNK,), i),
            pltpu.VMEM((_CHUNK // 128, 128), i),
            pltpu.VMEM((128,), i),
            pltpu.VMEM((128,), i),
            pltpu.VMEM((_NW * 128,), i),
            pltpu.VMEM((_BSL,), f),
            pltpu.VMEM((_BSL,), i),
            pltpu.VMEM_SHARED((_NW * 128,), i),
            pltpu.SemaphoreType.DMA,
        ],
    )(_sc_bucket_kernel)
    return kern(s, cls, x1, y1, x2, y2)


def _classnms_body(bs_ref, bx1_ref, by1_ref, bx2_ref, by2_ref, blin_ref,
                   ks_ref, klin_ref, kx1_ref, ky1_ref, kx2_ref, ky2_ref,
                   sel_ref, s_ref, ar_ref):
    x1v = bx1_ref[...]
    y1v = by1_ref[...]
    x2v = bx2_ref[...]
    y2v = by2_ref[...]
    s_ref[...] = bs_ref[...]
    ar_ref[...] = (x2v - x1v) * (y2v - y1v)
    rowi = jax.lax.broadcasted_iota(jnp.int32, (_C, 128), 0).astype(jnp.float32)
    lane = jax.lax.broadcasted_iota(jnp.int32, (1, 128), 1)
    laneoff = lane.astype(jnp.float32) * _MAX_WH
    zf = jnp.zeros((_C, 128), jnp.float32)
    ks_ref[...] = zf - 1e9
    klin_ref[...] = jnp.zeros((_C, 128), jnp.int32)
    kx1_ref[...] = zf
    ky1_ref[...] = zf
    kx2_ref[...] = zf
    ky2_ref[...] = zf

    def cond(carry):
        t, alive = carry
        return alive & (t < _C)

    def body(carry):
        t, _ = carry
        s = s_ref[...]
        m = jnp.max(s, axis=0, keepdims=True)
        okr = m > -1e8
        r = jnp.where(s >= m, rowi, 1e9)
        rif = jnp.min(r, axis=0, keepdims=True)
        hit = rowi == rif
        gx1 = jnp.sum(jnp.where(hit, x1v, 0.0), axis=0, keepdims=True)
        gy1 = jnp.sum(jnp.where(hit, y1v, 0.0), axis=0, keepdims=True)
        gx2 = jnp.sum(jnp.where(hit, x2v, 0.0), axis=0, keepdims=True)
        gy2 = jnp.sum(jnp.where(hit, y2v, 0.0), axis=0, keepdims=True)
        glin = jnp.sum(jnp.where(hit, blin_ref[...], 0), axis=0, keepdims=True)
        barea = (gx2 - gx1) * (gy2 - gy1)
        x1 = jnp.maximum(x1v, gx1)
        y1 = jnp.maximum(y1v, gy1)
        x2 = jnp.minimum(x2v, gx2)
        y2 = jnp.minimum(y2v, gy2)
        inter = jnp.maximum(x2 - x1, 0.0) * jnp.maximum(y2 - y1, 0.0)
        iou = inter / (ar_ref[...] + barea - inter + 1e-9)
        sup = (iou > _IOU_THRES) & okr
        s_ref[...] = jnp.where(sup | hit, -1e9, s)
        okf = jnp.where(okr, 1.0, 0.0)
        ks_ref[pl.ds(t, 1), :] = jnp.where(okr, m, -1e9)
        klin_ref[pl.ds(t, 1), :] = jnp.where(okr, glin, 0)
        kx1_ref[pl.ds(t, 1), :] = (gx1 - laneoff) * okf
        ky1_ref[pl.ds(t, 1), :] = (gy1 - laneoff) * okf
        kx2_ref[pl.ds(t, 1), :] = (gx2 - laneoff) * okf
        ky2_ref[pl.ds(t, 1), :] = (gy2 - laneoff) * okf
        return t + 1, jnp.any(okr)

    jax.lax.while_loop(cond, body, (jnp.int32(0), jnp.bool_(True)))

    ksi = jax.lax.bitcast_convert_type(ks_ref[...], jnp.int32)
    klv = klin_ref[...]
    lo0 = jax.lax.bitcast_convert_type(jnp.float32(_CONF_THRES), jnp.int32) - 1
    hi0 = jax.lax.bitcast_convert_type(jnp.float32(1.0), jnp.int32)

    def cnt_gt(x):
        return jnp.sum((ksi > x).astype(jnp.int32))

    k_total = cnt_gt(lo0)

    def bsearch(i, lh):
        lo, hi = lh
        mid = (lo + hi) // 2
        le = cnt_gt(mid) <= (_MAX_DET - 1)
        return jnp.where(le, lo, mid), jnp.where(le, mid, hi)

    lo_f, hi_f = jax.lax.fori_loop(0, 24, bsearch, (lo0, hi0))
    xstar = jnp.where(k_total <= (_MAX_DET - 1), lo0, hi_f)
    cnt1 = cnt_gt(xstar)
    need = _MAX_DET - cnt1
    eqm = ksi == xstar

    def cnt_lin(L):
        return jnp.sum((eqm & (klv < L)).astype(jnp.int32))

    def lsearch(i, lh):
        lo, hi = lh
        mid = (lo + hi) // 2
        ge = cnt_lin(mid) >= need
        return jnp.where(ge, lo, mid), jnp.where(ge, mid, hi)

    llo, lhi = jax.lax.fori_loop(0, 16, lsearch,
                                 (jnp.int32(-1), jnp.int32(_NPAD + 1)))
    sel_ref[...] = ((ksi > xstar) | (eqm & (klv < lhi))).astype(jnp.int32)


def _run_classnms(bs, bx1, by1, bx2, by2, blin):
    f = jnp.float32
    i = jnp.int32
    sh_f = jax.ShapeDtypeStruct((_C, 128), f)
    sh_i = jax.ShapeDtypeStruct((_C, 128), i)
    return pl.pallas_call(
        _classnms_body,
        out_shape=(sh_f, sh_i, sh_f, sh_f, sh_f, sh_f, sh_i),
        scratch_shapes=[pltpu.VMEM((_C, 128), f), pltpu.VMEM((_C, 128), f)],
    )(bs, bx1, by1, bx2, by2, blin)


def _sc_merge_kernel(ks_hbm, klin_hbm, sel_hbm, kx1_hbm, ky1_hbm,
                     kx2_hbm, ky2_hbm,
                     det_hbm, ccs_hbm, clin_hbm, cslot_hbm,
                     ks_v, klin_v, sel_v, pos_v, pos2_v, slotid_v, fill_f,
                     ccs_v, clin_v, mine_slot, mine_rank,
                     g1_v, g2_v, g3_v, g4_v, idx6_v, val6_v,
                     tot_v, shared_tot, sem):
    wid = jax.lax.axis_index("s")
    iota16 = jax.lax.broadcasted_iota(jnp.int32, (16,), 0)
    myslice = wid * _BSL

    for j in range(1008 // 16):
        fill_f[pl.ds(j * 16, 16)] = jnp.zeros((16,), jnp.float32)
    for k in range(_DETSL // 1008):
        pltpu.sync_copy(fill_f, det_hbm.at[pl.ds(wid * _DETSL + k * 1008, 1008)])
    for j in range(_CSH // 16):
        fill_f[pl.ds(j * 16, 16)] = jnp.full((16,), -1e9, jnp.float32)
    pltpu.sync_copy(fill_f.at[pl.ds(0, _CSH)], ccs_hbm.at[pl.ds(wid * _CSH, _CSH)])

    pltpu.sync_copy(ks_hbm.at[pl.ds(myslice, _BSL)], ks_v)
    pltpu.sync_copy(klin_hbm.at[pl.ds(myslice, _BSL)], klin_v)
    pltpu.sync_copy(sel_hbm.at[pl.ds(myslice, _BSL)], sel_v)

    def pbody(j, run):
        v = sel_v[pl.ds(j * 16, 16)]
        cs = jax.lax.cumsum(v, axis=0)
        pos_v[pl.ds(j * 16, 16)] = run + cs - v
        return run + jnp.sum(v)

    my_total = jax.lax.fori_loop(0, _BSL // 16, pbody, jnp.int32(0))
    tot_v[...] = jnp.where(iota16 == 0, my_total, 0)
    pltpu.sync_copy(tot_v, shared_tot.at[pl.ds(wid * 16, 16)])
    plsc.subcore_barrier()

    base = jnp.int32(0)
    m_tot = jnp.int32(0)
    for w in range(_NW):
        pltpu.sync_copy(shared_tot.at[pl.ds(w * 16, 16)], tot_v)
        tw = jnp.sum(jnp.where(iota16 == 0, tot_v[...], 0))
        base = base + tw * (wid > w).astype(jnp.int32)
        m_tot = m_tot + tw

    allmask = iota16 >= 0
    for j in range(_BSL // 16):
        slotid_v[pl.ds(j * 16, 16)] = myslice + j * 16 + iota16
        p = pos_v[pl.ds(j * 16, 16)] + base
        sv = sel_v[pl.ds(j * 16, 16)]
        rowv = jnp.zeros((16,), jnp.int32) + j // 8
        colv = (j % 8) * 16 + iota16
        plsc.store_scatter(pos2_v, [rowv, colv],
                           jnp.where(sv > 0, p, _DUMPC), mask=allmask)
    copies = []
    for k in range(_BSL // 128):
        idx = pos2_v.at[k]
        sl = pl.ds(k * 128, 128)
        copies.append(pltpu.async_copy(ks_v.at[sl], ccs_hbm.at[idx], sem))
        copies.append(pltpu.async_copy(klin_v.at[sl], clin_hbm.at[idx], sem))
        copies.append(pltpu.async_copy(slotid_v.at[sl], cslot_hbm.at[idx], sem))
    for cp in copies:
        cp.wait()
    plsc.subcore_barrier()

    pltpu.sync_copy(ccs_hbm.at[pl.ds(0, _CM)], ccs_v)
    pltpu.sync_copy(clin_hbm.at[pl.ds(0, _CM)], clin_v)
    pltpu.sync_copy(cslot_hbm.at[pl.ds(wid * _CSH, _CSH)], mine_slot)

    def ebody(e, c0):
        ce = _sc_ext(ccs_v, wid * _CSH + e, 0.0)
        le = _sc_ext(clin_v, wid * _CSH + e, 0)

        def jbody(j, acc):
            cj = ccs_v[pl.ds(j * 16, 16)]
            lj = clin_v[pl.ds(j * 16, 16)]
            better = (cj > ce) | ((cj == ce) & (lj < le))
            return acc + better.astype(jnp.int32)

        acc = jax.lax.fori_loop(0, _CM // 16, jbody, jnp.zeros((16,), jnp.int32))
        _sc_put(mine_rank, e, jnp.sum(acc), jnp.int32)
        return c0

    jax.lax.fori_loop(0, _CSH, ebody, jnp.int32(0))

    for j in range(_CSH // 16):
        msl = mine_slot[pl.ds(j * 16, 16)]
        mine_slot[pl.ds(j * 16, 16)] = jnp.minimum(jnp.maximum(msl, 0), _BUCK - 1)
    copies = [
        pltpu.async_copy(kx1_hbm.at[mine_slot], g1_v, sem),
        pltpu.async_copy(ky1_hbm.at[mine_slot], g2_v, sem),
        pltpu.async_copy(kx2_hbm.at[mine_slot], g3_v, sem),
        pltpu.async_copy(ky2_hbm.at[mine_slot], g4_v, sem),
    ]
    for cp in copies:
        cp.wait()
    for j in range(_CSH // 16):
        sl = pl.ds(j * 16, 16)
        eg = wid * _CSH + j * 16 + iota16
        validm = eg < m_tot
        rank = mine_rank[sl]
        clsv = mine_slot[sl] & 127
        sc = ccs_v[pl.ds(wid * _CSH + j * 16, 16)]
        vals = [g1_v[sl], g2_v[sl], g3_v[sl], g4_v[sl], sc,
                clsv.astype(jnp.float32)]
        for ch in range(6):
            idx = jnp.where(validm, rank * 128 + ch, _DUMPD)
            rowv = jnp.zeros((16,), jnp.int32) + (j * 6 + ch)
            plsc.store_scatter(idx6_v, [rowv, iota16], idx, mask=iota16 >= 0)
            plsc.store_scatter(val6_v, [rowv, iota16], vals[ch],
                               mask=iota16 >= 0)
    copies = []
    for k in range(_CSH // 16 * 6):
        copies.append(pltpu.async_copy(val6_v.at[k], det_hbm.at[idx6_v.at[k]],
                                       sem))
    for cp in copies:
        cp.wait()


def _run_merge(ks, klin, sel, kx1, ky1, kx2, ky2):
    mesh = plsc.VectorSubcoreMesh(core_axis_name="c", subcore_axis_name="s",
                                  num_cores=1)
    f = jnp.float32
    i = jnp.int32
    kern = functools.partial(
        pl.kernel,
        mesh=mesh,
        compiler_params=pltpu.CompilerParams(needs_layout_passes=False),
        out_type=(
            jax.ShapeDtypeStruct((_DETFLAT,), f),
            jax.ShapeDtypeStruct((_CMPAD,), f),
            jax.ShapeDtypeStruct((_CMPAD,), i),
            jax.ShapeDtypeStruct((_CMPAD,), i),
        ),
        scratch_types=[
            pltpu.VMEM((_BSL,), f),
            pltpu.VMEM((_BSL,), i),
            pltpu.VMEM((_BSL,), i),
            pltpu.VMEM((_BSL,), i),
            pltpu.VMEM((_BSL // 128, 128), i),
            pltpu.VMEM((_BSL,), i),
            pltpu.VMEM((1008,), f),
            pltpu.VMEM((_CM,), f),
            pltpu.VMEM((_CM,), i),
            pltpu.VMEM((_CSH,), i),
            pltpu.VMEM((_CSH,), i),
            pltpu.VMEM((_CSH,), f),
            pltpu.VMEM((_CSH,), f),
            pltpu.VMEM((_CSH,), f),
            pltpu.VMEM((_CSH,), f),
            pltpu.VMEM((_CSH // 16 * 6, 16), i),
            pltpu.VMEM((_CSH // 16 * 6, 16), f),
            pltpu.VMEM((16,), i),
            pltpu.VMEM_SHARED((_NW * 16,), i),
            pltpu.SemaphoreType.DMA,
        ],
    )(_sc_merge_kernel)
    return kern(ks, klin, sel, kx1, ky1, kx2, ky2)


def kernel(pred):
    p = pred[0]
    pt = jnp.transpose(p)
    pt = jnp.pad(pt, ((0, 0), (0, _NPAD - _N)))
    pt = pt.reshape(85, _R, 128)

    s, cls, x1, y1, x2, y2 = _run_prep(pt)
    bs, bx1, by1, bx2, by2, blin, whist = _run_bucket(
        s.reshape(_NPAD), cls.reshape(_NPAD), x1.reshape(_NPAD),
        y1.reshape(_NPAD), x2.reshape(_NPAD), y2.reshape(_NPAD))
    counts = jnp.sum(whist.reshape(_NW, 128), axis=0)
    overflow = jnp.max(counts[:127]) > _C

    def slow(_):
        return _run_full(pt)

    def fast(_):
        b2 = lambda a: a[:_BUCK].reshape(_C, 128)
        ks, klin, kx1, ky1, kx2, ky2, sel = _run_classnms(
            b2(bs), b2(bx1), b2(by1), b2(bx2), b2(by2), b2(blin))
        det, _, _, _ = _run_merge(
            ks.reshape(_BUCK), klin.reshape(_BUCK), sel.reshape(_BUCK),
            kx1.reshape(_BUCK), ky1.reshape(_BUCK),
            kx2.reshape(_BUCK), ky2.reshape(_BUCK))
        return det.reshape(_DETROWS, 128)[:_MAX_DET, :6]

    return jax.lax.cond(overflow, slow, fast, 0)
